# baseline (speedup 1.0000x reference)
import jax
import jax.numpy as jnp
from jax.experimental import pallas as pl

BATCH = 4096
HIST_LEN = 200
EXPAND_DIM = 64
BB = 128
IB = 200
N_IB = HIST_LEN // IB


def _body(tst_ref, et_ref, out_ref):
    ib = pl.program_id(1)
    tst = tst_ref[...]
    h, nb = tst.shape
    tsi = tst_ref[pl.ds(ib * IB, IB), :][None, :, :]
    tsj = tst[:, None, :]
    d = tsj - tsi
    jj = jax.lax.broadcasted_iota(jnp.int32, (h, IB, nb), 0)
    ii = jax.lax.broadcasted_iota(jnp.int32, (h, IB, nb), 1) + ib * IB
    less = (d < 0) | ((d == 0) & (jj < ii))
    rank = jnp.sum(less.astype(jnp.float32), axis=0)
    kk = jj.astype(jnp.float32)
    oh = (rank[None, :, :] == kk).astype(jnp.float32)
    et = jnp.broadcast_to(et_ref[0][None, :, :], (h, EXPAND_DIM, IB))
    part = jax.lax.dot_general(
        et, oh, (((2,), (1,)), ((0,), (0,))),
        preferred_element_type=jnp.float32,
    )

    @pl.when(ib == 0)
    def _init():
        out_ref[...] = part

    @pl.when(ib > 0)
    def _acc():
        out_ref[...] += part


@jax.jit
def kernel(ts, pos_embeddings):
    ts_t = ts.T
    et = pos_embeddings.T.reshape(EXPAND_DIM, N_IB, IB).transpose(1, 0, 2)
    grid = (BATCH // BB, N_IB)
    out_t = pl.pallas_call(
        _body,
        grid=grid,
        in_specs=[
            pl.BlockSpec((HIST_LEN, BB), lambda bb, ib: (0, bb)),
            pl.BlockSpec((1, EXPAND_DIM, IB), lambda bb, ib: (ib, 0, 0)),
        ],
        out_specs=pl.BlockSpec((HIST_LEN, EXPAND_DIM, BB),
                               lambda bb, ib: (0, 0, bb)),
        out_shape=jax.ShapeDtypeStruct((HIST_LEN, EXPAND_DIM, BATCH),
                                       jnp.float32),
    )(ts_t, et)
    return out_t.transpose(2, 0, 1)
